# baseline (speedup 1.0000x reference)
import functools

import jax
import jax.numpy as jnp
from jax import lax
from jax.experimental import pallas as pl
from jax.experimental.pallas import tpu as pltpu
from jax.experimental.pallas import tpu_sc as plsc

N_NODES = 10000
D_FEAT = 128
N_EDGES = 320000

_NC = 2
_NS = 16
_NW = _NC * _NS

_B_PER_W = N_EDGES // _NW
_CHUNK = 80
_N_CHUNKS = _B_PER_W // _CHUNK
_NBUF = 4
_N_MAIN = (_N_CHUNKS // _NBUF) * _NBUF
_N_TAIL = _N_CHUNKS - _N_MAIN

_mesh = plsc.VectorSubcoreMesh(core_axis_name="c", subcore_axis_name="s")


@functools.partial(
    pl.kernel,
    out_type=jax.ShapeDtypeStruct((N_EDGES, D_FEAT), jnp.float32),
    mesh=_mesh,
    scratch_types=[
        pltpu.VMEM_SHARED((N_NODES, D_FEAT), jnp.float32),
        pltpu.VMEM((_B_PER_W,), jnp.int32),
        [pltpu.VMEM((_CHUNK, D_FEAT), jnp.float32) for _ in range(_NBUF)],
        [pltpu.SemaphoreType.DMA for _ in range(_NBUF)],
        [pltpu.SemaphoreType.DMA for _ in range(_NBUF)],
        pltpu.SemaphoreType.DMA,
    ],
)
def _gather_kernel(x_hbm, idx_hbm, out_hbm, x_sh, idx_all, rows_v,
                   sem_g, sem_w, sem_st):
    sid = lax.axis_index("s")
    wid = sid * _NC + lax.axis_index("c")
    base = wid * _B_PER_W

    @pl.when(sid < 15)
    def _stage_a():
        r0 = sid * 624
        pltpu.async_copy(x_hbm.at[pl.ds(r0, 624)], x_sh.at[pl.ds(r0, 624)],
                         sem_st)

    @pl.when(sid == 15)
    def _stage_b():
        pltpu.async_copy(x_hbm.at[pl.ds(9360, 640)],
                         x_sh.at[pl.ds(9360, 640)], sem_st)

    pltpu.sync_copy(idx_hbm.at[pl.ds(base, _B_PER_W)], idx_all)

    def start_gather(chunk, b, src):
        idx_c = idx_all.at[pl.ds(chunk * _CHUNK, _CHUNK)]
        return pltpu.async_copy(src.at[idx_c], rows_v[b], sem_g[b])

    def out_slice(chunk):
        return out_hbm.at[pl.ds(base + chunk * _CHUNK, _CHUNK)]

    first = [start_gather(b, b, x_hbm) for b in range(_NBUF)]

    @pl.when(sid < 15)
    def _stage_a_wait():
        r0 = sid * 624
        pltpu.make_async_copy(x_hbm.at[pl.ds(r0, 624)],
                              x_sh.at[pl.ds(r0, 624)], sem_st).wait()

    @pl.when(sid == 15)
    def _stage_b_wait():
        pltpu.make_async_copy(x_hbm.at[pl.ds(9360, 640)],
                              x_sh.at[pl.ds(9360, 640)], sem_st).wait()

    plsc.subcore_barrier()

    for b in range(_NBUF):
        first[b].wait()
        pltpu.async_copy(rows_v[b], out_slice(b), sem_w[b])

    @pl.loop(_NBUF, _N_MAIN, step=_NBUF)
    def _chunk(i0):
        gathers = []
        for b in range(_NBUF):
            pltpu.make_async_copy(rows_v[b], out_slice(i0 + b),
                                  sem_w[b]).wait()
            gathers.append(start_gather(i0 + b, b, x_sh))
        for b in range(_NBUF):
            gathers[b].wait()
            pltpu.async_copy(rows_v[b], out_slice(i0 + b), sem_w[b])

    tail_gathers = []
    for b in range(_N_TAIL):
        pltpu.make_async_copy(rows_v[b], out_slice(_N_MAIN + b),
                              sem_w[b]).wait()
        tail_gathers.append(start_gather(_N_MAIN + b, b, x_sh))
    for b in range(_N_TAIL):
        tail_gathers[b].wait()
        pltpu.async_copy(rows_v[b], out_slice(_N_MAIN + b), sem_w[b])

    for b in range(_NBUF):
        chunk = _N_MAIN + b if b < _N_TAIL else _N_MAIN - _NBUF + b
        pltpu.make_async_copy(rows_v[b], out_slice(chunk), sem_w[b]).wait()


def kernel(x, idx):
    return _gather_kernel(x, idx.astype(jnp.int32))


# speedup vs baseline: 1.0094x; 1.0094x over previous
import functools

import jax
import jax.numpy as jnp
from jax import lax
from jax.experimental import pallas as pl
from jax.experimental.pallas import tpu as pltpu
from jax.experimental.pallas import tpu_sc as plsc

N_NODES = 10000
D_FEAT = 128
N_EDGES = 320000

_NC = 2
_NS = 16
_NW = _NC * _NS

_B_PER_W = N_EDGES // _NW
_CHUNK = 40
_N_CHUNKS = _B_PER_W // _CHUNK
_NBUF = 8
_N_MAIN = (_N_CHUNKS // _NBUF) * _NBUF
_N_TAIL = _N_CHUNKS - _N_MAIN

_mesh = plsc.VectorSubcoreMesh(core_axis_name="c", subcore_axis_name="s")


@functools.partial(
    pl.kernel,
    out_type=jax.ShapeDtypeStruct((N_EDGES, D_FEAT), jnp.float32),
    mesh=_mesh,
    scratch_types=[
        pltpu.VMEM_SHARED((N_NODES, D_FEAT), jnp.float32),
        pltpu.VMEM((_B_PER_W,), jnp.int32),
        [pltpu.VMEM((_CHUNK, D_FEAT), jnp.float32) for _ in range(_NBUF)],
        [pltpu.SemaphoreType.DMA for _ in range(_NBUF)],
        [pltpu.SemaphoreType.DMA for _ in range(_NBUF)],
        pltpu.SemaphoreType.DMA,
    ],
)
def _gather_kernel(x_hbm, idx_hbm, out_hbm, x_sh, idx_all, rows_v,
                   sem_g, sem_w, sem_st):
    sid = lax.axis_index("s")
    wid = sid * _NC + lax.axis_index("c")
    base = wid * _B_PER_W

    @pl.when(sid < 15)
    def _stage_a():
        r0 = sid * 624
        pltpu.async_copy(x_hbm.at[pl.ds(r0, 624)], x_sh.at[pl.ds(r0, 624)],
                         sem_st)

    @pl.when(sid == 15)
    def _stage_b():
        pltpu.async_copy(x_hbm.at[pl.ds(9360, 640)],
                         x_sh.at[pl.ds(9360, 640)], sem_st)

    pltpu.sync_copy(idx_hbm.at[pl.ds(base, _B_PER_W)], idx_all)

    def start_gather(chunk, b, src):
        idx_c = idx_all.at[pl.ds(chunk * _CHUNK, _CHUNK)]
        return pltpu.async_copy(src.at[idx_c], rows_v[b], sem_g[b])

    def out_slice(chunk):
        return out_hbm.at[pl.ds(base + chunk * _CHUNK, _CHUNK)]

    first = [start_gather(b, b, x_hbm) for b in range(_NBUF)]

    @pl.when(sid < 15)
    def _stage_a_wait():
        r0 = sid * 624
        pltpu.make_async_copy(x_hbm.at[pl.ds(r0, 624)],
                              x_sh.at[pl.ds(r0, 624)], sem_st).wait()

    @pl.when(sid == 15)
    def _stage_b_wait():
        pltpu.make_async_copy(x_hbm.at[pl.ds(9360, 640)],
                              x_sh.at[pl.ds(9360, 640)], sem_st).wait()

    plsc.subcore_barrier()

    for b in range(_NBUF):
        first[b].wait()
        pltpu.async_copy(rows_v[b], out_slice(b), sem_w[b])

    @pl.loop(_NBUF, _N_MAIN, step=_NBUF)
    def _chunk(i0):
        gathers = []
        for b in range(_NBUF):
            pltpu.make_async_copy(rows_v[b], out_slice(i0 + b),
                                  sem_w[b]).wait()
            gathers.append(start_gather(i0 + b, b, x_sh))
        for b in range(_NBUF):
            gathers[b].wait()
            pltpu.async_copy(rows_v[b], out_slice(i0 + b), sem_w[b])

    tail_gathers = []
    for b in range(_N_TAIL):
        pltpu.make_async_copy(rows_v[b], out_slice(_N_MAIN + b),
                              sem_w[b]).wait()
        tail_gathers.append(start_gather(_N_MAIN + b, b, x_sh))
    for b in range(_N_TAIL):
        tail_gathers[b].wait()
        pltpu.async_copy(rows_v[b], out_slice(_N_MAIN + b), sem_w[b])

    for b in range(_NBUF):
        chunk = _N_MAIN + b if b < _N_TAIL else _N_MAIN - _NBUF + b
        pltpu.make_async_copy(rows_v[b], out_slice(chunk), sem_w[b]).wait()


def kernel(x, idx):
    return _gather_kernel(x, idx.astype(jnp.int32))
